# baseline (speedup 1.0000x reference)
import functools

import jax
import jax.numpy as jnp
from jax import lax
from jax.experimental import pallas as pl
from jax.experimental.pallas import tpu as pltpu
from jax.experimental.pallas import tpu_sc as plsc

N = 10000
E = 320000
HV, HE, P, Q = 128, 64, 16, 16
D1 = HV + HE + P + Q
D2 = HE + P + Q

NC, NS = 2, 16
NW = NC * NS
GW = 128
EPAD = E + (-E) % (GW * NW)
SW = 128

EB = 2000
VB = 2000


def _node_body(hv_ref, p_ref, q_ref, wmh_ref, weu_ref, wev_ref,
               t1_ref, t2_ref):
    hv = hv_ref[...]
    a = jnp.dot(hv, wmh_ref[...], preferred_element_type=jnp.float32)
    bu = jnp.dot(hv, weu_ref[...], preferred_element_type=jnp.float32)
    bv = jnp.dot(hv, wev_ref[...], preferred_element_type=jnp.float32)
    pq = jnp.concatenate([p_ref[...], q_ref[...]], axis=1)
    t1_ref[...] = jnp.concatenate([a, bu, pq], axis=1)
    t2_ref[...] = jnp.concatenate([bv, pq], axis=1)


def _tc_node(hv_ftr, p_ftr, q_ftr, wm_hv, we_u, we_v):
    nsteps = N // VB
    full = lambda arr: pl.BlockSpec(arr.shape, lambda i: (0,) * arr.ndim)
    return pl.pallas_call(
        _node_body,
        grid=(nsteps,),
        in_specs=[
            pl.BlockSpec((VB, HV), lambda i: (i, 0)),
            pl.BlockSpec((VB, P), lambda i: (i, 0)),
            pl.BlockSpec((VB, Q), lambda i: (i, 0)),
            full(wm_hv), full(we_u), full(we_v),
        ],
        out_specs=[
            pl.BlockSpec((VB, D1), lambda i: (i, 0)),
            pl.BlockSpec((VB, D2), lambda i: (i, 0)),
        ],
        out_shape=[
            jax.ShapeDtypeStruct((N, D1), jnp.float32),
            jax.ShapeDtypeStruct((N, D2), jnp.float32),
        ],
    )(hv_ftr, p_ftr, q_ftr, wm_hv, we_u, we_v)


def _sc_gather_pair(t1, idx_src, t2, idx_dst):
    mesh = plsc.VectorSubcoreMesh(core_axis_name="c", subcore_axis_name="s")

    @functools.partial(
        pl.kernel,
        out_type=(jax.ShapeDtypeStruct((EPAD, D1), t1.dtype),
                  jax.ShapeDtypeStruct((EPAD, D2), t2.dtype)),
        mesh=mesh,
        compiler_params=pltpu.CompilerParams(use_tc_tiling_on_sc=False),
    )
    def k(t1_hbm, i1_hbm, t2_hbm, i2_hbm, o1_hbm, o2_hbm):
        def body1(i_vmem, o_vmem):
            pltpu.sync_copy(t1_hbm.at[i_vmem.at[0]], o_vmem)

        pltpu.emit_pipeline(
            body1,
            grid=(EPAD // GW,),
            in_specs=[pl.BlockSpec((1, GW), lambda i: (0, i))],
            out_specs=[pl.BlockSpec((GW, D1), lambda i: (i, 0))],
            core_axis_name=("c", "s"),
            dimension_semantics=(pltpu.PARALLEL,),
        )(i1_hbm, o1_hbm)

        def body2(i_vmem, o_vmem):
            pltpu.sync_copy(t2_hbm.at[i_vmem.at[0]], o_vmem)

        pltpu.emit_pipeline(
            body2,
            grid=(EPAD // GW,),
            in_specs=[pl.BlockSpec((1, GW), lambda i: (0, i))],
            out_specs=[pl.BlockSpec((GW, D2), lambda i: (i, 0))],
            core_axis_name=("c", "s"),
            dimension_semantics=(pltpu.PARALLEL,),
        )(i2_hbm, o2_hbm)

    return k(t1, idx_src.reshape(1, EPAD), t2, idx_dst.reshape(1, EPAD))


def _sc_segment_sum(msg, dst3d):
    n_rows = dst3d.shape[0]
    n_full = N // SW
    tail = N - n_full * SW
    n_chunks = n_full + (1 if tail else 0)
    chunk_iters = (n_chunks + NS - 1) // NS
    mesh = plsc.VectorSubcoreMesh(core_axis_name="c", subcore_axis_name="s")
    max_iters = (n_rows + NW - 1) // NW

    @functools.partial(
        pl.kernel,
        out_type=jax.ShapeDtypeStruct((NC, N, HV), jnp.float32),
        mesh=mesh,
        scratch_types=[
            pltpu.VMEM((1, SW), jnp.int32),
            pltpu.VMEM((SW, HV), jnp.float32),
            pltpu.VMEM_SHARED((N, HV), jnp.float32),
        ],
    )
    def k(msg_hbm, dst_hbm, out_hbm, idx_v, buf_v, acc_sh):
        cid = lax.axis_index("c")
        sid = lax.axis_index("s")
        wid = sid * NC + cid

        @pl.loop(0, SW)
        def _(r):
            @pl.loop(0, HV, step=16)
            def _(col):
                buf_v[r, pl.ds(col, 16)] = jnp.zeros((16,), jnp.float32)

        @pl.loop(0, chunk_iters)
        def _(j):
            c = sid + j * NS

            @pl.when(c < n_full)
            def _():
                pltpu.sync_copy(buf_v, acc_sh.at[pl.ds(c * SW, SW)])

            @pl.when(c == n_full)
            def _():
                pltpu.sync_copy(buf_v.at[pl.ds(0, tail)],
                                acc_sh.at[pl.ds(n_full * SW, tail)])

        plsc.subcore_barrier()

        @pl.loop(0, max_iters)
        def _(j):
            r = wid + j * NW

            @pl.when(r < n_rows)
            def _():
                pltpu.sync_copy(dst_hbm.at[r], idx_v)
                pltpu.sync_copy(msg_hbm.at[pl.ds(r * SW, SW)], buf_v)
                pltpu.sync_copy(buf_v, acc_sh.at[idx_v.at[0]], add=True)

        plsc.subcore_barrier()

        @pl.loop(0, chunk_iters)
        def _(j):
            c = sid + j * NS

            @pl.when(c < n_full)
            def _():
                pltpu.sync_copy(acc_sh.at[pl.ds(c * SW, SW)],
                                out_hbm.at[cid, pl.ds(c * SW, SW)])

            @pl.when(c == n_full)
            def _():
                pltpu.sync_copy(acc_sh.at[pl.ds(n_full * SW, tail)],
                                out_hbm.at[cid, pl.ds(n_full * SW, tail)])

    return k(msg, dst3d)


def _edge_body(x1_ref, x2_ref, he_ref,
               wpq_ref, wall_ref, wih_ref,
               bm_ref, be_ref, bih_ref, bhh_ref,
               msg_ref, heo_ref):
    x1 = x1_ref[...]
    x2 = x2_ref[...]
    he = he_ref[...]
    dpq = x1[:, HV + HE:] - x2[:, HE:]

    hemm = jnp.dot(he, wall_ref[...], preferred_element_type=jnp.float32)

    msg = (x1[:, :HV]
           + jnp.dot(dpq, wpq_ref[...], preferred_element_type=jnp.float32)
           + hemm[:, :HV] + bm_ref[...])
    msg_ref[...] = jnp.maximum(msg, 0.0)

    me = x1[:, HV:HV + HE] + x2[:, :HE] + hemm[:, HV:HV + HE] + be_ref[...]
    me = jnp.maximum(me, 0.0)

    gi = jnp.dot(me, wih_ref[...], preferred_element_type=jnp.float32) + bih_ref[...]
    gh = hemm[:, HV + HE:] + bhh_ref[...]
    r = jax.nn.sigmoid(gi[:, :HE] + gh[:, :HE])
    z = jax.nn.sigmoid(gi[:, HE:2 * HE] + gh[:, HE:2 * HE])
    n = jnp.tanh(gi[:, 2 * HE:] + r * gh[:, 2 * HE:])
    heo_ref[...] = (1.0 - z) * n + z * he


def _tc_edge(x1, x2, he_ftr, wpq, wall, wih, bm2, be2, bih2, bhh2):
    nsteps = E // EB
    full = lambda arr: pl.BlockSpec(arr.shape, lambda i: (0,) * arr.ndim)
    return pl.pallas_call(
        _edge_body,
        grid=(nsteps,),
        in_specs=[
            pl.BlockSpec((EB, D1), lambda i: (i, 0)),
            pl.BlockSpec((EB, D2), lambda i: (i, 0)),
            pl.BlockSpec((EB, HE), lambda i: (i, 0)),
            full(wpq), full(wall), full(wih),
            full(bm2), full(be2), full(bih2), full(bhh2),
        ],
        out_specs=[
            pl.BlockSpec((EB, HV), lambda i: (i, 0)),
            pl.BlockSpec((EB, HE), lambda i: (i, 0)),
        ],
        out_shape=[
            jax.ShapeDtypeStruct((E, HV), jnp.float32),
            jax.ShapeDtypeStruct((E, HE), jnp.float32),
        ],
    )(x1, x2, he_ftr, wpq, wall, wih, bm2, be2, bih2, bhh2)


def _vertex_body(p0_ref, p1_ref, hv_ref, wih_ref, whh_ref, bih_ref, bhh_ref,
                 out_ref):
    mv = p0_ref[0] + p1_ref[0]
    hv = hv_ref[...]
    gi = jnp.dot(mv, wih_ref[...], preferred_element_type=jnp.float32) + bih_ref[...]
    gh = jnp.dot(hv, whh_ref[...], preferred_element_type=jnp.float32) + bhh_ref[...]
    r = jax.nn.sigmoid(gi[:, :HV] + gh[:, :HV])
    z = jax.nn.sigmoid(gi[:, HV:2 * HV] + gh[:, HV:2 * HV])
    n = jnp.tanh(gi[:, 2 * HV:] + r * gh[:, 2 * HV:])
    out_ref[...] = (1.0 - z) * n + z * hv


def _tc_vertex(parts, hv_ftr, wih_v_t, whh_v_t, bih_v2, bhh_v2):
    nsteps = N // VB
    full = lambda arr: pl.BlockSpec(arr.shape, lambda i: (0,) * arr.ndim)
    return pl.pallas_call(
        _vertex_body,
        grid=(nsteps,),
        in_specs=[
            pl.BlockSpec((1, VB, HV), lambda i: (0, i, 0)),
            pl.BlockSpec((1, VB, HV), lambda i: (0, i, 0)),
            pl.BlockSpec((VB, HV), lambda i: (i, 0)),
            full(wih_v_t), full(whh_v_t), full(bih_v2), full(bhh_v2),
        ],
        out_specs=pl.BlockSpec((VB, HV), lambda i: (i, 0)),
        out_shape=jax.ShapeDtypeStruct((N, HV), jnp.float32),
    )(parts[:1], parts[1:], hv_ftr, wih_v_t, whh_v_t, bih_v2, bhh_v2)


def kernel(hv_ftr, he_ftr, p_ftr, q_ftr, Wm, bm, We, be,
           Wih_v, Whh_v, bih_v, bhh_v, Wih_e, Whh_e, bih_e, bhh_e,
           edge_index):
    wm_hv = Wm[:, :HV].T
    wm_he = Wm[:, HV:HV + HE].T
    wm_pq = Wm[:, HV + HE:].T
    we_u = We[:, :HV].T
    we_v = We[:, HV:2 * HV].T
    we_e = We[:, 2 * HV:].T
    wall = jnp.concatenate([wm_he, we_e, Whh_e.T], axis=1)

    t1, t2 = _tc_node(hv_ftr, p_ftr, q_ftr, wm_hv, we_u, we_v)

    idx_s = jnp.pad(edge_index[0], (0, EPAD - E))
    idx_d = jnp.pad(edge_index[1], (0, EPAD - E))
    x1, x2 = _sc_gather_pair(t1, idx_s, t2, idx_d)

    msg, he_out = _tc_edge(
        x1, x2, he_ftr, wm_pq, wall, Wih_e.T,
        bm.reshape(1, -1), be.reshape(1, -1),
        bih_e.reshape(1, -1), bhh_e.reshape(1, -1))

    dst3d = edge_index[1].reshape(E // SW, 1, SW)
    parts = _sc_segment_sum(msg, dst3d)

    hv_out = _tc_vertex(parts, hv_ftr, Wih_v.T, Whh_v.T,
                        bih_v.reshape(1, -1), bhh_v.reshape(1, -1))
    return (hv_out, he_out)
